# baseline (speedup 1.0000x reference)
import functools

import jax
import jax.numpy as jnp
from jax import lax
from jax.experimental import pallas as pl
from jax.experimental.pallas import tpu as pltpu
from jax.experimental.pallas import tpu_sc as plsc

_B, _N, _D, _K = 8, 1024, 3, 20
_C1, _C2, _C3 = 32, 128, 256
_NCLS = 40
_NK = _N * _K
_NW = 32
_CHUNK = 128


def _sc_gather(table, idx, c):
    nch = idx.shape[1]
    rpw = nch * _CHUNK
    mesh = plsc.VectorSubcoreMesh(core_axis_name="c", subcore_axis_name="s")

    @functools.partial(
        pl.kernel,
        mesh=mesh,
        out_type=jax.ShapeDtypeStruct((_NW * rpw, c), jnp.float32),
        scratch_types=[
            pltpu.VMEM((nch, _CHUNK), jnp.int32),
            pltpu.VMEM((2, _CHUNK, c), jnp.float32),
            pltpu.SemaphoreType.DMA,
        ],
    )
    def gather_kernel(table_hbm, idx_hbm, out_hbm, idx_v, rows_v, sem):
        wid = lax.axis_index("s") * 2 + lax.axis_index("c")
        pltpu.sync_copy(idx_hbm.at[wid], idx_v)
        base = wid * rpw
        pltpu.make_async_copy(
            table_hbm.at[idx_v.at[0]], rows_v.at[0], sem).start()

        def body(j, carry):
            @pl.when(j + 1 < nch)
            def _():
                pltpu.make_async_copy(
                    table_hbm.at[idx_v.at[j + 1]], rows_v.at[(j + 1) % 2],
                    sem).start()
            pltpu.make_async_copy(
                table_hbm.at[idx_v.at[j]], rows_v.at[j % 2], sem).wait()
            pltpu.sync_copy(rows_v.at[j % 2],
                            out_hbm.at[pl.ds(base + j * _CHUNK, _CHUNK)])
            return carry

        lax.fori_loop(0, nch, body, 0)

    return gather_kernel(table, idx)


def _softmax0(w2d):
    def body(w_ref, o_ref):
        w = w_ref[...]
        m = jnp.max(w, axis=0, keepdims=True)
        e = jnp.exp(w - m)
        o_ref[...] = e / jnp.sum(e, axis=0, keepdims=True)

    return pl.pallas_call(
        body, out_shape=jax.ShapeDtypeStruct(w2d.shape, jnp.float32))(w2d)


def _wfm_folded(g, wbig, cout, nblk):
    cpad = g.shape[-1]

    def body(g_ref, w_ref, o_ref):
        acc = jnp.dot(g_ref[0, 0], w_ref[0],
                      preferred_element_type=jnp.float32)
        for k in range(1, _K):
            acc = acc + jnp.dot(g_ref[k, 0], w_ref[k],
                                preferred_element_type=jnp.float32)
        s = [acc[:, d * cout:(d + 1) * cout] for d in range(_D)]
        inv = 1.0 / (jnp.sqrt(s[0] * s[0] + s[1] * s[1] + s[2] * s[2])
                     + 1e-8)
        for d in range(_D):
            o_ref[0, :, d * cout:(d + 1) * cout] = s[d] * inv

    return pl.pallas_call(
        body,
        grid=(_B, _N // nblk),
        in_specs=[
            pl.BlockSpec((_K, 1, nblk, cpad), lambda b, n: (0, b, n, 0)),
            pl.BlockSpec((_K, cpad, _D * cout), lambda b, n: (0, 0, 0)),
        ],
        out_specs=pl.BlockSpec((1, nblk, _D * cout), lambda b, n: (b, n, 0)),
        out_shape=jax.ShapeDtypeStruct((_B, _N, _D * cout), jnp.float32),
    )(g, wbig)


def _wfm_perd(g, w, cin, cout, nblk):
    nb = g.shape[1]

    def body(g_ref, w_ref, o_ref):
        accs = []
        for d in range(_D):
            acc = None
            for k in range(_K):
                gk = g_ref[k, 0, :, d * cin:(d + 1) * cin]
                t = jnp.dot(gk, w_ref[k], preferred_element_type=jnp.float32)
                acc = t if acc is None else acc + t
            accs.append(acc)
        inv = 1.0 / (jnp.sqrt(accs[0] * accs[0] + accs[1] * accs[1]
                              + accs[2] * accs[2]) + 1e-8)
        for d in range(_D):
            o_ref[0, :, d * cout:(d + 1) * cout] = accs[d] * inv

    return pl.pallas_call(
        body,
        grid=(nb, _N // nblk),
        in_specs=[
            pl.BlockSpec((_K, 1, nblk, _D * cin), lambda b, n: (0, b, n, 0)),
            pl.BlockSpec((_K, cin, cout), lambda b, n: (0, 0, 0)),
        ],
        out_specs=pl.BlockSpec((1, nblk, _D * cout), lambda b, n: (b, n, 0)),
        out_shape=jax.ShapeDtypeStruct((nb, _N, _D * cout), jnp.float32),
    )(g, w)


def _head(h, w_last, b_last):
    def body(h_ref, w_ref, b_ref, o_ref):
        hs = [h_ref[0, :, d * _C3:(d + 1) * _C3] for d in range(_D)]
        ms = [jnp.mean(hd, axis=0, keepdims=True) for hd in hs]
        inv = 1.0 / (jnp.sqrt(ms[0] * ms[0] + ms[1] * ms[1]
                              + ms[2] * ms[2]) + 1e-8)
        ms = [m * inv for m in ms]
        cos = hs[0] * ms[0] + hs[1] * ms[1] + hs[2] * ms[2]
        cos = jnp.clip(cos, -1.0 + 1e-6, 1.0 - 1e-6)
        dist = lax.atan2(jnp.sqrt(1.0 - cos * cos), cos)
        feat = jnp.mean(dist, axis=0, keepdims=True)
        o_ref[0] = jnp.dot(feat, w_ref[...],
                           preferred_element_type=jnp.float32) + b_ref[...]

    nb = h.shape[0]
    out = pl.pallas_call(
        body,
        grid=(nb,),
        in_specs=[
            pl.BlockSpec((1, _N, _D * _C3), lambda b: (b, 0, 0)),
            pl.BlockSpec((_C3, _NCLS), lambda b: (0, 0)),
            pl.BlockSpec((1, _NCLS), lambda b: (0, 0)),
        ],
        out_specs=pl.BlockSpec((1, 1, _NCLS), lambda b: (b, 0, 0)),
        out_shape=jax.ShapeDtypeStruct((nb, 1, _NCLS), jnp.float32),
    )(h, w_last, b_last.reshape(1, _NCLS))
    return out.reshape(nb, _NCLS)


def kernel(x, neighborhood_matrix, w1, w2, w3, W_last, b_last):
    idx = neighborhood_matrix.astype(jnp.int32)
    idx_km = (idx.transpose(2, 0, 1)
              + (jnp.arange(_B) * _N)[None, :, None])
    flat_idx = idx_km.reshape(_NW, -1, _CHUNK)
    flat_idx_lo = idx_km[:, :_B // 2].reshape(_NW, -1, _CHUNK)
    flat_idx_hi = idx_km[:, _B // 2:].reshape(_NW, -1, _CHUNK)

    ws1 = _softmax0(w1.reshape(_K, _C1))
    ws2 = _softmax0(w2.reshape(_K * _C1, _C2)).reshape(_K, _C1, _C2)
    ws3 = _softmax0(w3.reshape(_K * _C2, _C3)).reshape(_K, _C2, _C3)

    w1big = jnp.zeros((_K, 128, _D * _C1), jnp.float32)
    w2big = jnp.zeros((_K, 128, _D * _C2), jnp.float32)
    for d in range(_D):
        w1big = w1big.at[:, d, d * _C1:(d + 1) * _C1].set(ws1)
        w2big = w2big.at[:, d * _C1:(d + 1) * _C1,
                         d * _C2:(d + 1) * _C2].set(ws2)

    table1 = jnp.pad(x.reshape(_B * _N, _D), ((0, 0), (0, 128 - _D)))
    g1 = _sc_gather(table1, flat_idx, 128).reshape(_K, _B, _N, 128)
    h1 = _wfm_folded(g1, w1big, _C1, 256)

    table2 = jnp.pad(h1.reshape(_B * _N, _D * _C1),
                     ((0, 0), (0, 128 - _D * _C1)))
    g2 = _sc_gather(table2, flat_idx, 128).reshape(_K, _B, _N, 128)
    h2 = _wfm_folded(g2, w2big, _C2, 256)

    table3 = h2.reshape(_B * _N, _D * _C2)
    outs = []
    for fi in (flat_idx_lo, flat_idx_hi):
        g3 = _sc_gather(table3, fi, _D * _C2)
        g3 = g3.reshape(_K, _B // 2, _N, _D * _C2)
        h3 = _wfm_perd(g3, ws3, _C2, _C3, 256)
        outs.append(_head(h3, W_last, b_last))
    return jnp.concatenate(outs, axis=0)


# speedup vs baseline: 1.0254x; 1.0254x over previous
import functools

import jax
import jax.numpy as jnp
from jax import lax
from jax.experimental import pallas as pl
from jax.experimental.pallas import tpu as pltpu
from jax.experimental.pallas import tpu_sc as plsc

_B, _N, _D, _K = 8, 1024, 3, 20
_C1, _C2, _C3 = 32, 128, 256
_NCLS = 40
_NK = _N * _K
_NW = 32
_CHUNK = 128


def _sc_gather(table, idx, row_shape, dtype=jnp.float32):
    nch = idx.shape[1]
    rpw = nch * _CHUNK
    mesh = plsc.VectorSubcoreMesh(core_axis_name="c", subcore_axis_name="s")

    @functools.partial(
        pl.kernel,
        mesh=mesh,
        out_type=jax.ShapeDtypeStruct((_NW * rpw,) + row_shape, dtype),
        scratch_types=[
            pltpu.VMEM((nch, _CHUNK), jnp.int32),
            pltpu.VMEM((2, _CHUNK) + row_shape, dtype),
            pltpu.SemaphoreType.DMA,
        ],
    )
    def gather_kernel(table_hbm, idx_hbm, out_hbm, idx_v, rows_v, sem):
        wid = lax.axis_index("s") * 2 + lax.axis_index("c")
        pltpu.sync_copy(idx_hbm.at[wid], idx_v)
        base = wid * rpw
        pltpu.make_async_copy(
            table_hbm.at[idx_v.at[0]], rows_v.at[0], sem).start()

        def body(j, carry):
            @pl.when(j + 1 < nch)
            def _():
                pltpu.make_async_copy(
                    table_hbm.at[idx_v.at[j + 1]], rows_v.at[(j + 1) % 2],
                    sem).start()
            pltpu.make_async_copy(
                table_hbm.at[idx_v.at[j]], rows_v.at[j % 2], sem).wait()
            pltpu.sync_copy(rows_v.at[j % 2],
                            out_hbm.at[pl.ds(base + j * _CHUNK, _CHUNK)])
            return carry

        lax.fori_loop(0, nch, body, 0)

    return gather_kernel(table, idx)


def _softmax0(w2d):
    def body(w_ref, o_ref):
        w = w_ref[...]
        m = jnp.max(w, axis=0, keepdims=True)
        e = jnp.exp(w - m)
        o_ref[...] = e / jnp.sum(e, axis=0, keepdims=True)

    return pl.pallas_call(
        body, out_shape=jax.ShapeDtypeStruct(w2d.shape, jnp.float32))(w2d)


def _wfm_folded(g, wbig, cout, nblk):
    cpad = g.shape[-1]
    nb = g.shape[1]

    def body(g_ref, w_ref, o_ref):
        acc = jnp.dot(g_ref[0, 0], w_ref[0],
                      preferred_element_type=jnp.float32)
        for k in range(1, _K):
            acc = acc + jnp.dot(g_ref[k, 0], w_ref[k],
                                preferred_element_type=jnp.float32)
        s = [acc[:, d * cout:(d + 1) * cout] for d in range(_D)]
        inv = 1.0 / (jnp.sqrt(s[0] * s[0] + s[1] * s[1] + s[2] * s[2])
                     + 1e-8)
        for d in range(_D):
            o_ref[0, :, d * cout:(d + 1) * cout] = s[d] * inv

    return pl.pallas_call(
        body,
        grid=(nb, _N // nblk),
        in_specs=[
            pl.BlockSpec((_K, 1, nblk, cpad), lambda b, n: (0, b, n, 0)),
            pl.BlockSpec((_K, cpad, _D * cout), lambda b, n: (0, 0, 0)),
        ],
        out_specs=pl.BlockSpec((1, nblk, _D * cout), lambda b, n: (b, n, 0)),
        out_shape=jax.ShapeDtypeStruct((nb, _N, _D * cout), jnp.float32),
    )(g, wbig)


def _wfm_perd(g, w, cin, cout, nblk):
    nb = g.shape[1]

    def body(g_ref, w_ref, o_ref):
        accs = []
        for d in range(_D):
            acc = None
            for k in range(_K):
                gk = g_ref[k, 0, :, d * cin:(d + 1) * cin]
                t = jnp.dot(gk, w_ref[k], preferred_element_type=jnp.float32)
                acc = t if acc is None else acc + t
            accs.append(acc)
        inv = 1.0 / (jnp.sqrt(accs[0] * accs[0] + accs[1] * accs[1]
                              + accs[2] * accs[2]) + 1e-8)
        for d in range(_D):
            o_ref[0, :, d * cout:(d + 1) * cout] = accs[d] * inv

    return pl.pallas_call(
        body,
        grid=(nb, _N // nblk),
        in_specs=[
            pl.BlockSpec((_K, 1, nblk, _D * cin), lambda b, n: (0, b, n, 0)),
            pl.BlockSpec((_K, cin, cout), lambda b, n: (0, 0, 0)),
        ],
        out_specs=pl.BlockSpec((1, nblk, _D * cout), lambda b, n: (b, n, 0)),
        out_shape=jax.ShapeDtypeStruct((nb, _N, _D * cout), jnp.float32),
    )(g, w)


def _head(h, w_last, b_last):
    def body(h_ref, w_ref, b_ref, o_ref):
        hs = [h_ref[0, :, d * _C3:(d + 1) * _C3] for d in range(_D)]
        ms = [jnp.mean(hd, axis=0, keepdims=True) for hd in hs]
        inv = 1.0 / (jnp.sqrt(ms[0] * ms[0] + ms[1] * ms[1]
                              + ms[2] * ms[2]) + 1e-8)
        ms = [m * inv for m in ms]
        cos = hs[0] * ms[0] + hs[1] * ms[1] + hs[2] * ms[2]
        cos = jnp.clip(cos, -1.0 + 1e-6, 1.0 - 1e-6)
        dist = lax.atan2(jnp.sqrt(1.0 - cos * cos), cos)
        feat = jnp.mean(dist, axis=0, keepdims=True)
        o_ref[0] = jnp.dot(feat, w_ref[...],
                           preferred_element_type=jnp.float32) + b_ref[...]

    nb = h.shape[0]
    out = pl.pallas_call(
        body,
        grid=(nb,),
        in_specs=[
            pl.BlockSpec((1, _N, _D * _C3), lambda b: (b, 0, 0)),
            pl.BlockSpec((_C3, _NCLS), lambda b: (0, 0)),
            pl.BlockSpec((1, _NCLS), lambda b: (0, 0)),
        ],
        out_specs=pl.BlockSpec((1, 1, _NCLS), lambda b: (b, 0, 0)),
        out_shape=jax.ShapeDtypeStruct((nb, 1, _NCLS), jnp.float32),
    )(h, w_last, b_last.reshape(1, _NCLS))
    return out.reshape(nb, _NCLS)


def kernel(x, neighborhood_matrix, w1, w2, w3, W_last, b_last):
    idx = neighborhood_matrix.astype(jnp.int32)
    idx_km = (idx.transpose(2, 0, 1)
              + (jnp.arange(_B) * _N)[None, :, None])
    flat_idx = idx_km.reshape(_NW, -1, _CHUNK)

    ws1 = _softmax0(w1.reshape(_K, _C1))
    ws2 = _softmax0(w2.reshape(_K * _C1, _C2)).reshape(_K, _C1, _C2)
    ws3 = _softmax0(w3.reshape(_K * _C2, _C3)).reshape(_K, _C2, _C3)

    w1big = jnp.zeros((_K, 128, _D * _C1), jnp.float32)
    w2big = jnp.zeros((_K, 128, _D * _C2), jnp.float32)
    for d in range(_D):
        w1big = w1big.at[:, d, d * _C1:(d + 1) * _C1].set(ws1)
        w2big = w2big.at[:, d * _C1:(d + 1) * _C1,
                         d * _C2:(d + 1) * _C2].set(ws2)

    table1 = jnp.pad(x.reshape(_B * _N, _D), ((0, 0), (0, 128 - _D)))
    g1 = _sc_gather(table1, flat_idx, (128,)).reshape(_K, _B, _N, 128)
    h1 = _wfm_folded(g1, w1big, _C1, 256)

    table2 = jnp.pad(h1.reshape(_B * _N, _D * _C1),
                     ((0, 0), (0, 128 - _D * _C1)))
    g2 = _sc_gather(table2, flat_idx, (128,)).reshape(_K, _B, _N, 128)
    h2 = _wfm_folded(g2, w2big, _C2, 256)

    g3 = _sc_gather(h2.reshape(_B * _N, _D * _C2), flat_idx, (_D * _C2,))
    g3 = g3.reshape(_K, _B, _N, _D * _C2)
    h3 = _wfm_perd(g3, ws3, _C2, _C3, 256)
    return _head(h3, W_last, b_last)


# speedup vs baseline: 1.0751x; 1.0484x over previous
import functools

import jax
import jax.numpy as jnp
from jax import lax
from jax.experimental import pallas as pl
from jax.experimental.pallas import tpu as pltpu
from jax.experimental.pallas import tpu_sc as plsc

_B, _N, _D, _K = 8, 1024, 3, 20
_C1, _C2, _C3 = 32, 128, 256
_NCLS = 40
_NK = _N * _K
_NW = 32
_CHUNK = 128


def _sc_gather(table, idx, row_shape, dtype=jnp.float32):
    nch = idx.shape[1]
    rpw = nch * _CHUNK
    mesh = plsc.VectorSubcoreMesh(core_axis_name="c", subcore_axis_name="s")

    @functools.partial(
        pl.kernel,
        mesh=mesh,
        out_type=jax.ShapeDtypeStruct((_NW * rpw,) + row_shape, dtype),
        scratch_types=[
            pltpu.VMEM((nch, _CHUNK), jnp.int32),
            pltpu.VMEM((2, _CHUNK) + row_shape, dtype),
            pltpu.SemaphoreType.DMA,
        ],
    )
    def gather_kernel(table_hbm, idx_hbm, out_hbm, idx_v, rows_v, sem):
        wid = lax.axis_index("s") * 2 + lax.axis_index("c")
        pltpu.sync_copy(idx_hbm.at[wid], idx_v)
        base = wid * rpw
        pltpu.make_async_copy(
            table_hbm.at[idx_v.at[0]], rows_v.at[0], sem).start()

        def body(j, carry):
            @pl.when(j + 1 < nch)
            def _():
                pltpu.make_async_copy(
                    table_hbm.at[idx_v.at[j + 1]], rows_v.at[(j + 1) % 2],
                    sem).start()
            pltpu.make_async_copy(
                table_hbm.at[idx_v.at[j]], rows_v.at[j % 2], sem).wait()
            pltpu.sync_copy(rows_v.at[j % 2],
                            out_hbm.at[pl.ds(base + j * _CHUNK, _CHUNK)])
            return carry

        lax.fori_loop(0, nch, body, 0)

    return gather_kernel(table, idx)


def _softmax0(w2d):
    def body(w_ref, o_ref):
        w = w_ref[...]
        m = jnp.max(w, axis=0, keepdims=True)
        e = jnp.exp(w - m)
        o_ref[...] = e / jnp.sum(e, axis=0, keepdims=True)

    return pl.pallas_call(
        body, out_shape=jax.ShapeDtypeStruct(w2d.shape, jnp.float32))(w2d)


def _wfm_folded(g, wbig, cout, nblk):
    cpad = g.shape[-1]
    nb = g.shape[1]

    def body(g_ref, w_ref, o_ref):
        acc = jnp.dot(g_ref[0, 0], w_ref[0],
                      preferred_element_type=jnp.float32)
        for k in range(1, _K):
            acc = acc + jnp.dot(g_ref[k, 0], w_ref[k],
                                preferred_element_type=jnp.float32)
        s = [acc[:, d * cout:(d + 1) * cout] for d in range(_D)]
        inv = 1.0 / (jnp.sqrt(s[0] * s[0] + s[1] * s[1] + s[2] * s[2])
                     + 1e-8)
        for d in range(_D):
            o_ref[0, :, d * cout:(d + 1) * cout] = s[d] * inv

    return pl.pallas_call(
        body,
        grid=(nb, _N // nblk),
        in_specs=[
            pl.BlockSpec((_K, 1, nblk, cpad), lambda b, n: (0, b, n, 0)),
            pl.BlockSpec((_K, cpad, _D * cout), lambda b, n: (0, 0, 0)),
        ],
        out_specs=pl.BlockSpec((1, nblk, _D * cout), lambda b, n: (b, n, 0)),
        out_shape=jax.ShapeDtypeStruct((nb, _N, _D * cout), jnp.float32),
    )(g, wbig)


def _wfm_perd(g, w, cin, cout, nblk):
    nb = g.shape[1]

    def body(g_ref, w_ref, o_ref):
        accs = []
        for d in range(_D):
            acc = None
            for k in range(_K):
                gk = g_ref[k, 0, :, d * cin:(d + 1) * cin]
                t = jnp.dot(gk, w_ref[k], preferred_element_type=jnp.float32)
                acc = t if acc is None else acc + t
            accs.append(acc)
        inv = 1.0 / (jnp.sqrt(accs[0] * accs[0] + accs[1] * accs[1]
                              + accs[2] * accs[2]) + 1e-8)
        for d in range(_D):
            o_ref[0, :, d * cout:(d + 1) * cout] = accs[d] * inv

    return pl.pallas_call(
        body,
        grid=(nb, _N // nblk),
        in_specs=[
            pl.BlockSpec((_K, 1, nblk, _D * cin), lambda b, n: (0, b, n, 0)),
            pl.BlockSpec((_K, cin, cout), lambda b, n: (0, 0, 0)),
        ],
        out_specs=pl.BlockSpec((1, nblk, _D * cout), lambda b, n: (b, n, 0)),
        out_shape=jax.ShapeDtypeStruct((nb, _N, _D * cout), jnp.float32),
    )(g, w)


def _head(h, w_last, b_last):
    def body(h_ref, w_ref, b_ref, o_ref):
        hs = [h_ref[0, :, d * _C3:(d + 1) * _C3] for d in range(_D)]
        ms = [jnp.mean(hd, axis=0, keepdims=True) for hd in hs]
        inv = 1.0 / (jnp.sqrt(ms[0] * ms[0] + ms[1] * ms[1]
                              + ms[2] * ms[2]) + 1e-8)
        ms = [m * inv for m in ms]
        cos = hs[0] * ms[0] + hs[1] * ms[1] + hs[2] * ms[2]
        cos = jnp.clip(cos, -1.0 + 1e-6, 1.0 - 1e-6)
        dist = lax.atan2(jnp.sqrt(1.0 - cos * cos), cos)
        feat = jnp.mean(dist, axis=0, keepdims=True)
        o_ref[0] = jnp.dot(feat, w_ref[...],
                           preferred_element_type=jnp.float32) + b_ref[...]

    nb = h.shape[0]
    out = pl.pallas_call(
        body,
        grid=(nb,),
        in_specs=[
            pl.BlockSpec((1, _N, _D * _C3), lambda b: (b, 0, 0)),
            pl.BlockSpec((_C3, _NCLS), lambda b: (0, 0)),
            pl.BlockSpec((1, _NCLS), lambda b: (0, 0)),
        ],
        out_specs=pl.BlockSpec((1, 1, _NCLS), lambda b: (b, 0, 0)),
        out_shape=jax.ShapeDtypeStruct((nb, 1, _NCLS), jnp.float32),
    )(h, w_last, b_last.reshape(1, _NCLS))
    return out.reshape(nb, _NCLS)


def kernel(x, neighborhood_matrix, w1, w2, w3, W_last, b_last):
    idx = neighborhood_matrix.astype(jnp.int32)
    idx_km = (idx.transpose(2, 0, 1)
              + (jnp.arange(_B) * _N)[None, :, None])
    flat_idx = idx_km.reshape(_NW, -1, _CHUNK)

    ws1 = _softmax0(w1.reshape(_K, _C1))
    ws2 = _softmax0(w2.reshape(_K * _C1, _C2)).reshape(_K, _C1, _C2)
    ws3 = _softmax0(w3.reshape(_K * _C2, _C3)).reshape(_K, _C2, _C3)

    w1big = jnp.zeros((_K, 128, _D * _C1), jnp.float32)
    w2big = jnp.zeros((_K, 128, _D * _C2), jnp.float32)
    for d in range(_D):
        w1big = w1big.at[:, d, d * _C1:(d + 1) * _C1].set(ws1)
        w2big = w2big.at[:, d * _C1:(d + 1) * _C1,
                         d * _C2:(d + 1) * _C2].set(ws2)

    table1 = jnp.pad(x.reshape(_B * _N, _D), ((0, 0), (0, 128 - _D)))
    g1 = _sc_gather(table1, flat_idx, (128,)).reshape(_K, _B, _N, 128)
    h1 = _wfm_folded(g1, w1big, _C1, 512)

    table2 = jnp.pad(h1.reshape(_B * _N, _D * _C1),
                     ((0, 0), (0, 128 - _D * _C1)))
    g2 = _sc_gather(table2, flat_idx, (128,)).reshape(_K, _B, _N, 128)
    h2 = _wfm_folded(g2, w2big, _C2, 512)

    g3 = _sc_gather(h2.reshape(_B * _N, _D * _C2), flat_idx, (_D * _C2,))
    g3 = g3.reshape(_K, _B, _N, _D * _C2)
    h3 = _wfm_perd(g3, ws3, _C2, _C3, 512)
    return _head(h3, W_last, b_last)


# speedup vs baseline: 1.0872x; 1.0113x over previous
import functools

import jax
import jax.numpy as jnp
from jax import lax
from jax.experimental import pallas as pl
from jax.experimental.pallas import tpu as pltpu
from jax.experimental.pallas import tpu_sc as plsc

_B, _N, _D, _K = 8, 1024, 3, 20
_C1, _C2, _C3 = 32, 128, 256
_NCLS = 40
_NK = _N * _K
_NW = 32
_CHUNK = 128


def _sc_gather(table, idx, row_shape, dtype=jnp.float32):
    nch = idx.shape[1]
    rpw = nch * _CHUNK
    mesh = plsc.VectorSubcoreMesh(core_axis_name="c", subcore_axis_name="s")

    @functools.partial(
        pl.kernel,
        mesh=mesh,
        out_type=jax.ShapeDtypeStruct((_NW * rpw,) + row_shape, dtype),
        scratch_types=[
            pltpu.VMEM((nch, _CHUNK), jnp.int32),
            pltpu.VMEM((2, _CHUNK) + row_shape, dtype),
            pltpu.SemaphoreType.DMA,
        ],
    )
    def gather_kernel(table_hbm, idx_hbm, out_hbm, idx_v, rows_v, sem):
        wid = lax.axis_index("s") * 2 + lax.axis_index("c")
        pltpu.sync_copy(idx_hbm.at[wid], idx_v)
        base = wid * rpw
        pltpu.make_async_copy(
            table_hbm.at[idx_v.at[0]], rows_v.at[0], sem).start()

        def body(j, carry):
            @pl.when(j + 1 < nch)
            def _():
                pltpu.make_async_copy(
                    table_hbm.at[idx_v.at[j + 1]], rows_v.at[(j + 1) % 2],
                    sem).start()
            pltpu.make_async_copy(
                table_hbm.at[idx_v.at[j]], rows_v.at[j % 2], sem).wait()
            pltpu.sync_copy(rows_v.at[j % 2],
                            out_hbm.at[pl.ds(base + j * _CHUNK, _CHUNK)])
            return carry

        lax.fori_loop(0, nch, body, 0)

    return gather_kernel(table, idx)


def _softmax0(w2d):
    def body(w_ref, o_ref):
        w = w_ref[...]
        m = jnp.max(w, axis=0, keepdims=True)
        e = jnp.exp(w - m)
        o_ref[...] = e / jnp.sum(e, axis=0, keepdims=True)

    return pl.pallas_call(
        body, out_shape=jax.ShapeDtypeStruct(w2d.shape, jnp.float32))(w2d)


def _wfm_folded(g, wbig, cout, nblk):
    cpad = g.shape[-1]
    nb = g.shape[1]

    def body(g_ref, w_ref, o_ref):
        acc = jnp.dot(g_ref[0, 0], w_ref[0],
                      preferred_element_type=jnp.float32)
        for k in range(1, _K):
            acc = acc + jnp.dot(g_ref[k, 0], w_ref[k],
                                preferred_element_type=jnp.float32)
        s = [acc[:, d * cout:(d + 1) * cout] for d in range(_D)]
        inv = 1.0 / (jnp.sqrt(s[0] * s[0] + s[1] * s[1] + s[2] * s[2])
                     + 1e-8)
        for d in range(_D):
            o_ref[0, :, d * cout:(d + 1) * cout] = s[d] * inv

    return pl.pallas_call(
        body,
        grid=(nb, _N // nblk),
        in_specs=[
            pl.BlockSpec((_K, 1, nblk, cpad), lambda b, n: (0, b, n, 0)),
            pl.BlockSpec((_K, cpad, _D * cout), lambda b, n: (0, 0, 0)),
        ],
        out_specs=pl.BlockSpec((1, nblk, _D * cout), lambda b, n: (b, n, 0)),
        out_shape=jax.ShapeDtypeStruct((nb, _N, _D * cout), jnp.float32),
    )(g, wbig)


def _wfm_perd(g, w, cin, cout, nblk):
    nb = g.shape[1]

    def body(g_ref, w_ref, o_ref):
        accs = []
        for d in range(_D):
            acc = None
            for k in range(_K):
                gk = g_ref[k, 0, :, d * cin:(d + 1) * cin]
                t = jnp.dot(gk, w_ref[k], preferred_element_type=jnp.float32)
                acc = t if acc is None else acc + t
            accs.append(acc)
        inv = 1.0 / (jnp.sqrt(accs[0] * accs[0] + accs[1] * accs[1]
                              + accs[2] * accs[2]) + 1e-8)
        for d in range(_D):
            o_ref[0, :, d * cout:(d + 1) * cout] = accs[d] * inv

    return pl.pallas_call(
        body,
        grid=(nb, _N // nblk),
        in_specs=[
            pl.BlockSpec((_K, 1, nblk, _D * cin), lambda b, n: (0, b, n, 0)),
            pl.BlockSpec((_K, cin, cout), lambda b, n: (0, 0, 0)),
        ],
        out_specs=pl.BlockSpec((1, nblk, _D * cout), lambda b, n: (b, n, 0)),
        out_shape=jax.ShapeDtypeStruct((nb, _N, _D * cout), jnp.float32),
    )(g, w)


def _head(h, w_last, b_last):
    def body(h_ref, w_ref, b_ref, o_ref):
        hs = [h_ref[0, :, d * _C3:(d + 1) * _C3] for d in range(_D)]
        ms = [jnp.mean(hd, axis=0, keepdims=True) for hd in hs]
        inv = 1.0 / (jnp.sqrt(ms[0] * ms[0] + ms[1] * ms[1]
                              + ms[2] * ms[2]) + 1e-8)
        ms = [m * inv for m in ms]
        cos = hs[0] * ms[0] + hs[1] * ms[1] + hs[2] * ms[2]
        cos = jnp.clip(cos, -1.0 + 1e-6, 1.0 - 1e-6)
        dist = lax.atan2(jnp.sqrt(1.0 - cos * cos), cos)
        feat = jnp.mean(dist, axis=0, keepdims=True)
        o_ref[0] = jnp.dot(feat, w_ref[...],
                           preferred_element_type=jnp.float32) + b_ref[...]

    nb = h.shape[0]
    out = pl.pallas_call(
        body,
        grid=(nb,),
        in_specs=[
            pl.BlockSpec((1, _N, _D * _C3), lambda b: (b, 0, 0)),
            pl.BlockSpec((_C3, _NCLS), lambda b: (0, 0)),
            pl.BlockSpec((1, _NCLS), lambda b: (0, 0)),
        ],
        out_specs=pl.BlockSpec((1, 1, _NCLS), lambda b: (b, 0, 0)),
        out_shape=jax.ShapeDtypeStruct((nb, 1, _NCLS), jnp.float32),
    )(h, w_last, b_last.reshape(1, _NCLS))
    return out.reshape(nb, _NCLS)


def kernel(x, neighborhood_matrix, w1, w2, w3, W_last, b_last):
    idx = neighborhood_matrix.astype(jnp.int32)
    idx_km = (idx.transpose(2, 0, 1)
              + (jnp.arange(_B) * _N)[None, :, None])
    flat_idx = idx_km.reshape(_NW, -1, _CHUNK)

    ws1 = _softmax0(w1.reshape(_K, _C1))
    ws2 = _softmax0(w2.reshape(_K * _C1, _C2)).reshape(_K, _C1, _C2)
    ws3 = _softmax0(w3.reshape(_K * _C2, _C3)).reshape(_K, _C2, _C3)

    w1big = jnp.zeros((_K, 128, _D * _C1), jnp.float32)
    w2big = jnp.zeros((_K, 128, _D * _C2), jnp.float32)
    for d in range(_D):
        w1big = w1big.at[:, d, d * _C1:(d + 1) * _C1].set(ws1)
        w2big = w2big.at[:, d * _C1:(d + 1) * _C1,
                         d * _C2:(d + 1) * _C2].set(ws2)

    table1 = jnp.pad(x.reshape(_B * _N, _D), ((0, 0), (0, 128 - _D)))
    g1 = _sc_gather(table1, flat_idx, (128,)).reshape(_K, _B, _N, 128)
    h1 = _wfm_folded(g1, w1big, _C1, 1024)

    table2 = jnp.pad(h1.reshape(_B * _N, _D * _C1),
                     ((0, 0), (0, 128 - _D * _C1)))
    g2 = _sc_gather(table2, flat_idx, (128,)).reshape(_K, _B, _N, 128)
    h2 = _wfm_folded(g2, w2big, _C2, 1024)

    g3 = _sc_gather(h2.reshape(_B * _N, _D * _C2), flat_idx, (_D * _C2,))
    g3 = g3.reshape(_K, _B, _N, _D * _C2)
    h3 = _wfm_perd(g3, ws3, _C2, _C3, 512)
    return _head(h3, W_last, b_last)
